# baseline (speedup 1.0000x reference)
import functools

import jax
import jax.numpy as jnp
from jax import lax
from jax.experimental import pallas as pl
from jax.experimental.pallas import tpu as pltpu
from jax.experimental.pallas import tpu_sc as plsc

N_FRAMES = 128
N_SAMPLES = 32768
WT_SIZE = 1024
NC = 2
NS = 16
L = 16
NW = NC * NS
CHUNK = N_SAMPLES // NW
GROUPS = CHUNK // L
TAIL0 = N_SAMPLES - N_FRAMES

_mesh = plsc.VectorSubcoreMesh(
    core_axis_name="c", subcore_axis_name="s", num_cores=NC, num_subcores=NS
)


@functools.partial(
    pl.kernel,
    out_type=jax.ShapeDtypeStruct((N_SAMPLES,), jnp.float32),
    mesh=_mesh,
    compiler_params=pltpu.CompilerParams(
        needs_layout_passes=False,
        skip_device_barrier=True,
        disable_bounds_checks=True,
        disable_semaphore_checks=True,
    ),
    scratch_types=[
        pltpu.VMEM((N_FRAMES,), jnp.float32),
        pltpu.VMEM((WT_SIZE,), jnp.float32),
        pltpu.VMEM((N_FRAMES,), jnp.float32),
        pltpu.VMEM((GROUPS,), jnp.float32),
        pltpu.VMEM((GROUPS,), jnp.float32),
        pltpu.VMEM((GROUPS,), jnp.float32),
        pltpu.VMEM((CHUNK,), jnp.float32),
    ],
)
def _sc_kernel(p_hbm, wt_hbm, out_hbm, p_v, wt_v, e_v, offs_v, a_v, b_v, o_v):
    wid = lax.axis_index("s") * NC + lax.axis_index("c")
    pltpu.sync_copy(p_hbm, p_v)
    pltpu.sync_copy(wt_hbm, wt_v)

    lanes = lax.iota(jnp.int32, L)

    carry = jnp.float32(0.0)
    for v in range(N_FRAMES // L):
        xv = p_v[pl.ds(v * L, L)]
        cv = plsc.cumsum(xv)
        e_v[pl.ds(v * L, L)] = (cv - xv) + carry
        carry = carry + jnp.sum(xv)

    base = wid * CHUNK
    for v in range(GROUPS // L):
        i0 = base + (v * L + lanes) * L
        t = jnp.maximum(i0 - N_FRAMES, 0)
        k = t >> 8
        r = (t & 255).astype(jnp.float32)
        head = i0 < N_FRAMES
        tail = i0 >= TAIL0
        k1 = jnp.minimum(k + 1, N_FRAMES - 1)
        pk = plsc.load_gather(p_v, [k])
        pk1 = plsc.load_gather(p_v, [k1])
        ek = plsc.load_gather(e_v, [k])
        dp = pk1 - pk
        g = r * r * jnp.float32(1.0 / 512.0)
        off_main = 128.0 * (2.0 * ek + pk) + (r - g) * pk + g * pk1
        offs = jnp.where(head, i0.astype(jnp.float32) * pk, off_main)
        s = jnp.where(head | tail, jnp.float32(0.0), jnp.float32(1.0 / 256.0))
        b = (r + 0.5) * s
        offs_v[pl.ds(v * L, L)] = offs
        a_v[pl.ds(v * L, L)] = pk + b * dp
        b_v[pl.ds(v * L, L)] = s * dp

    ladder = (lanes + 1).astype(jnp.float32)
    tri = ((lanes * (lanes + 1)) >> 1).astype(jnp.float32)

    @plsc.parallel_loop(0, GROUPS, unroll=8)
    def body(j):
        jv = jnp.full((L,), j, jnp.int32)
        offs = plsc.load_gather(offs_v, [jv])
        av = plsc.load_gather(a_v, [jv])
        bv = plsc.load_gather(b_v, [jv])
        phase = offs + ladder * av + tri * bv
        m = jnp.where(phase < 0.0, phase + 1024.0, phase)
        wlo = m.astype(jnp.int32)
        wfr = m - wlo.astype(jnp.float32)
        wlo_g = jnp.minimum(wlo, WT_SIZE - 1)
        whi_g = (wlo + 1) & (WT_SIZE - 1)
        vlo = plsc.load_gather(wt_v, [wlo_g])
        vhi = plsc.load_gather(wt_v, [whi_g])
        o_v[pl.ds(j * L, L)] = vlo + wfr * (vhi - vlo)

    pltpu.sync_copy(o_v, out_hbm.at[pl.ds(wid * CHUNK, CHUNK)])


def kernel(x, p, wavetable):
    del x
    return _sc_kernel(p, wavetable)


# speedup vs baseline: 1.0135x; 1.0135x over previous
import functools

import jax
import jax.numpy as jnp
from jax import lax
from jax.experimental import pallas as pl
from jax.experimental.pallas import tpu as pltpu
from jax.experimental.pallas import tpu_sc as plsc

N_FRAMES = 128
N_SAMPLES = 32768
WT_SIZE = 1024
NC = 2
NS = 16
L = 16
NW = NC * NS
CHUNK = N_SAMPLES // NW
GROUPS = CHUNK // L
TAIL0 = N_SAMPLES - N_FRAMES

_mesh = plsc.VectorSubcoreMesh(
    core_axis_name="c", subcore_axis_name="s", num_cores=NC, num_subcores=NS
)


@functools.partial(
    pl.kernel,
    out_type=jax.ShapeDtypeStruct((N_SAMPLES,), jnp.float32),
    mesh=_mesh,
    compiler_params=pltpu.CompilerParams(
        needs_layout_passes=False,
        skip_device_barrier=True,
        disable_bounds_checks=True,
        disable_semaphore_checks=True,
    ),
    scratch_types=[
        pltpu.VMEM((N_FRAMES,), jnp.float32),
        pltpu.VMEM((WT_SIZE,), jnp.float32),
        pltpu.VMEM((N_FRAMES,), jnp.float32),
        pltpu.VMEM((GROUPS,), jnp.float32),
        pltpu.VMEM((GROUPS,), jnp.float32),
        pltpu.VMEM((GROUPS,), jnp.float32),
        pltpu.VMEM((CHUNK,), jnp.float32),
        pltpu.SemaphoreType.DMA,
    ],
)
def _sc_kernel(p_hbm, wt_hbm, out_hbm, p_v, wt_v, e_v, offs_v, a_v, b_v, o_v,
               wt_sem):
    wid = lax.axis_index("s") * NC + lax.axis_index("c")
    pltpu.sync_copy(p_hbm, p_v)
    wt_dma = pltpu.async_copy(wt_hbm, wt_v, wt_sem)

    lanes = lax.iota(jnp.int32, L)

    carry = jnp.float32(0.0)
    for v in range(N_FRAMES // L):
        xv = p_v[pl.ds(v * L, L)]
        cv = plsc.cumsum(xv)
        e_v[pl.ds(v * L, L)] = (cv - xv) + carry
        carry = carry + jnp.sum(xv)

    base = wid * CHUNK
    for v in range(GROUPS // L):
        i0 = base + (v * L + lanes) * L
        t = jnp.maximum(i0 - N_FRAMES, 0)
        k = t >> 8
        r = (t & 255).astype(jnp.float32)
        head = i0 < N_FRAMES
        tail = i0 >= TAIL0
        k1 = jnp.minimum(k + 1, N_FRAMES - 1)
        pk = plsc.load_gather(p_v, [k])
        pk1 = plsc.load_gather(p_v, [k1])
        ek = plsc.load_gather(e_v, [k])
        dp = pk1 - pk
        g = r * r * jnp.float32(1.0 / 512.0)
        off_main = 128.0 * (2.0 * ek + pk) + (r - g) * pk + g * pk1
        offs = jnp.where(head, i0.astype(jnp.float32) * pk, off_main)
        s = jnp.where(head | tail, jnp.float32(0.0), jnp.float32(1.0 / 256.0))
        b = (r + 0.5) * s
        offs_v[pl.ds(v * L, L)] = offs
        a_v[pl.ds(v * L, L)] = pk + b * dp
        b_v[pl.ds(v * L, L)] = s * dp

    ladder = (lanes + 1).astype(jnp.float32)
    tri = ((lanes * (lanes + 1)) >> 1).astype(jnp.float32)
    wt_dma.wait()

    @plsc.parallel_loop(0, GROUPS, unroll=4)
    def body(j):
        jv = jnp.full((L,), j, jnp.int32)
        offs = plsc.load_gather(offs_v, [jv])
        av = plsc.load_gather(a_v, [jv])
        bv = plsc.load_gather(b_v, [jv])
        phase = offs + ladder * av + tri * bv
        m = jnp.where(phase < 0.0, phase + 1024.0, phase)
        wlo = m.astype(jnp.int32)
        wfr = m - wlo.astype(jnp.float32)
        wlo_g = jnp.minimum(wlo, WT_SIZE - 1)
        whi_g = (wlo + 1) & (WT_SIZE - 1)
        vlo = plsc.load_gather(wt_v, [wlo_g])
        vhi = plsc.load_gather(wt_v, [whi_g])
        o_v[pl.ds(j * L, L)] = vlo + wfr * (vhi - vlo)

    pltpu.sync_copy(o_v, out_hbm.at[pl.ds(wid * CHUNK, CHUNK)])


def kernel(x, p, wavetable):
    del x
    return _sc_kernel(p, wavetable)
